# baseline (speedup 1.0000x reference)
import jax
import jax.numpy as jnp
from jax.experimental import pallas as pl

_D = 64
_F = 128
_E = 4
_STB = 2048


def _moe_body(x_ref, wr_ref, w1_ref, w3_ref, w2_ref, o_ref):
    xb = x_ref[0]
    dn = (((0,), (0,)), ((), ()))
    lg = jax.lax.dot_general(wr_ref[...], xb, dn,
                             preferred_element_type=jnp.float32)

    row = jax.lax.broadcasted_iota(jnp.int32, lg.shape, 0)
    neg_inf = jnp.float32(-jnp.inf)
    m1 = jnp.max(lg, axis=0, keepdims=True)
    i1 = jnp.min(jnp.where(lg == m1, row, _E), axis=0, keepdims=True)
    mask1 = row == i1
    lg2 = jnp.where(mask1, neg_inf, lg)
    m2 = jnp.max(lg2, axis=0, keepdims=True)
    i2 = jnp.min(jnp.where(lg2 == m2, row, _E), axis=0, keepdims=True)
    mask2 = row == i2
    e2 = jnp.exp(m2 - m1)
    g1 = 1.0 / (1.0 + e2)
    g2 = 1.0 - g1
    gt = jnp.where(mask1, g1, 0.0) + jnp.where(mask2, g2, 0.0)

    acc = jnp.zeros((_D, xb.shape[1]), jnp.float32)
    for e in range(_E):
        h1 = jax.lax.dot_general(w1_ref[e], xb, dn,
                                 preferred_element_type=jnp.float32)
        h3 = jax.lax.dot_general(w3_ref[e], xb, dn,
                                 preferred_element_type=jnp.float32)
        h = h1 * (0.5 * jnp.tanh(0.5 * h1) + 0.5) * h3
        y = jax.lax.dot_general(w2_ref[e], h, dn,
                                preferred_element_type=jnp.float32)
        acc = acc + y * gt[e:e + 1]
    o_ref[0] = acc


def kernel(x, Wr, W1, W2, W3):
    b, s, d = x.shape
    sb = s // _STB
    xt = jnp.transpose(x, (0, 2, 1))

    out = pl.pallas_call(
        _moe_body,
        grid=(b * sb,),
        in_specs=[
            pl.BlockSpec((1, d, _STB), lambda i: (i // sb, 0, i % sb)),
            pl.BlockSpec((_D, _E), lambda i: (0, 0)),
            pl.BlockSpec((_E, _D, _F), lambda i: (0, 0, 0)),
            pl.BlockSpec((_E, _D, _F), lambda i: (0, 0, 0)),
            pl.BlockSpec((_E, _F, _D), lambda i: (0, 0, 0)),
        ],
        out_specs=pl.BlockSpec((1, d, _STB), lambda i: (i // sb, 0, i % sb)),
        out_shape=jax.ShapeDtypeStruct((b, d, s), jnp.float32),
    )(xt, Wr, W1, W3, W2)
    return jnp.transpose(out, (0, 2, 1))


# speedup vs baseline: 1.2276x; 1.2276x over previous
import jax
import jax.numpy as jnp
from jax.experimental import pallas as pl

_D = 64
_F = 128
_E = 4
_STB = 4096


def _moe_body(x_ref, wr_ref, w13_ref, w2_ref, o_ref):
    xb = x_ref[0]
    dn = (((0,), (0,)), ((), ()))
    lg = jax.lax.dot_general(wr_ref[...], xb, dn,
                             preferred_element_type=jnp.float32)

    row = jax.lax.broadcasted_iota(jnp.int32, lg.shape, 0)
    neg_inf = jnp.float32(-jnp.inf)
    m1 = jnp.max(lg, axis=0, keepdims=True)
    i1 = jnp.min(jnp.where(lg == m1, row, _E), axis=0, keepdims=True)
    mask1 = row == i1
    lg2 = jnp.where(mask1, neg_inf, lg)
    m2 = jnp.max(lg2, axis=0, keepdims=True)
    i2 = jnp.min(jnp.where(lg2 == m2, row, _E), axis=0, keepdims=True)
    mask2 = row == i2
    e2 = jnp.exp(m2 - m1)
    g1 = 1.0 / (1.0 + e2)
    g2 = 1.0 - g1
    gt = jnp.where(mask1, g1, 0.0) + jnp.where(mask2, g2, 0.0)

    acc = jnp.zeros((_D, xb.shape[1]), jnp.float32)
    for e in range(_E):
        h13 = jax.lax.dot_general(w13_ref[e], xb, dn,
                                  preferred_element_type=jnp.float32)
        h1 = h13[:_F]
        h3 = h13[_F:]
        h = h1 * (0.5 * jnp.tanh(0.5 * h1) + 0.5) * h3
        y = jax.lax.dot_general(w2_ref[e], h, dn,
                                preferred_element_type=jnp.float32)
        acc = acc + y * gt[e:e + 1]
    o_ref[0] = acc


def kernel(x, Wr, W1, W2, W3):
    b, s, d = x.shape
    sb = s // _STB
    xt = jnp.transpose(x, (0, 2, 1))
    w13 = jnp.concatenate([W1, W3], axis=2)

    out = pl.pallas_call(
        _moe_body,
        grid=(b * sb,),
        in_specs=[
            pl.BlockSpec((1, d, _STB), lambda i: (i // sb, 0, i % sb)),
            pl.BlockSpec((_D, _E), lambda i: (0, 0)),
            pl.BlockSpec((_E, _D, 2 * _F), lambda i: (0, 0, 0)),
            pl.BlockSpec((_E, _F, _D), lambda i: (0, 0, 0)),
        ],
        out_specs=pl.BlockSpec((1, d, _STB), lambda i: (i // sb, 0, i % sb)),
        out_shape=jax.ShapeDtypeStruct((b, d, s), jnp.float32),
    )(xt, Wr, w13, W2)
    return jnp.transpose(out, (0, 2, 1))


# speedup vs baseline: 1.2586x; 1.0252x over previous
import jax
import jax.numpy as jnp
from jax.experimental import pallas as pl
from jax.experimental.pallas import tpu as pltpu

_D = 64
_F = 128
_E = 4
_STB = 4096


def _moe_body(x_ref, wr_ref, w13_ref, w2_ref, o_ref):
    xb = x_ref[0]
    dn = (((0,), (0,)), ((), ()))
    lg = jax.lax.dot_general(wr_ref[...], xb, dn,
                             preferred_element_type=jnp.float32)

    row = jax.lax.broadcasted_iota(jnp.int32, lg.shape, 0)
    neg_inf = jnp.float32(-jnp.inf)
    m1 = jnp.max(lg, axis=0, keepdims=True)
    i1 = jnp.min(jnp.where(lg == m1, row, _E), axis=0, keepdims=True)
    mask1 = row == i1
    lg2 = jnp.where(mask1, neg_inf, lg)
    m2 = jnp.max(lg2, axis=0, keepdims=True)
    i2 = jnp.min(jnp.where(lg2 == m2, row, _E), axis=0, keepdims=True)
    mask2 = row == i2
    e2 = jnp.exp(m2 - m1)
    g1 = 1.0 / (1.0 + e2)
    g2 = 1.0 - g1
    gt = jnp.where(mask1, g1, 0.0) + jnp.where(mask2, g2, 0.0)

    acc = jnp.zeros((_D, xb.shape[1]), jnp.float32)
    for e in range(_E):
        h13 = jax.lax.dot_general(w13_ref[e], xb, dn,
                                  preferred_element_type=jnp.float32)
        h1 = h13[:_F]
        h3 = h13[_F:]
        b13 = (0.5 * h1) * h3
        h = b13 * jnp.tanh(0.5 * h1) + b13
        y = jax.lax.dot_general(w2_ref[e], h, dn,
                                preferred_element_type=jnp.float32)
        acc = acc + y * gt[e:e + 1]
    o_ref[0] = acc


def kernel(x, Wr, W1, W2, W3):
    b, s, d = x.shape
    sb = s // _STB
    xt = jnp.transpose(x, (0, 2, 1))
    w13 = jnp.concatenate([W1, W3], axis=2)

    out = pl.pallas_call(
        _moe_body,
        grid=(b * sb,),
        in_specs=[
            pl.BlockSpec((1, d, _STB), lambda i: (i // sb, 0, i % sb)),
            pl.BlockSpec((_D, _E), lambda i: (0, 0)),
            pl.BlockSpec((_E, _D, 2 * _F), lambda i: (0, 0, 0)),
            pl.BlockSpec((_E, _F, _D), lambda i: (0, 0, 0)),
        ],
        out_specs=pl.BlockSpec((1, d, _STB), lambda i: (i // sb, 0, i % sb)),
        out_shape=jax.ShapeDtypeStruct((b, d, s), jnp.float32),
        compiler_params=pltpu.CompilerParams(
            dimension_semantics=("parallel",),
        ),
    )(xt, Wr, w13, W2)
    return jnp.transpose(out, (0, 2, 1))


# speedup vs baseline: 1.3933x; 1.1070x over previous
import jax
import jax.numpy as jnp
from jax.experimental import pallas as pl
from jax.experimental.pallas import tpu as pltpu

_D = 64
_F = 128
_E = 4
_STB = 4096


def _moe_body(x_ref, wrt_ref, w13_ref, w2t_ref, o_ref):
    xb = x_ref[0]
    dn = (((0,), (0,)), ((), ()))
    dnt = (((1,), (0,)), ((), ()))
    lg = jax.lax.dot_general(wrt_ref[...], xb, dnt,
                             preferred_element_type=jnp.float32)

    row = jax.lax.broadcasted_iota(jnp.int32, lg.shape, 0)
    neg_inf = jnp.float32(-jnp.inf)
    m1 = jnp.max(lg, axis=0, keepdims=True)
    i1 = jnp.min(jnp.where(lg == m1, row, _E), axis=0, keepdims=True)
    mask1 = row == i1
    lg2 = jnp.where(mask1, neg_inf, lg)
    m2 = jnp.max(lg2, axis=0, keepdims=True)
    i2 = jnp.min(jnp.where(lg2 == m2, row, _E), axis=0, keepdims=True)
    mask2 = row == i2
    e2 = jnp.exp(m2 - m1)
    g1 = 1.0 / (1.0 + e2)
    g2 = 1.0 - g1
    gt = jnp.where(mask1, g1, 0.0) + jnp.where(mask2, g2, 0.0)

    acc = jnp.zeros((_D, xb.shape[1]), jnp.float32)
    for e in range(_E):
        h13 = jax.lax.dot_general(w13_ref[e], xb, dn,
                                  preferred_element_type=jnp.float32)
        hh = h13[:_F]
        h3 = h13[_F:]
        b13 = hh * h3
        h = b13 * jnp.tanh(hh) + b13
        y = jax.lax.dot_general(w2t_ref[e], h, dnt,
                                preferred_element_type=jnp.float32)
        acc = acc + y * gt[e:e + 1]
    o_ref[0] = acc


def kernel(x, Wr, W1, W2, W3):
    b, s, d = x.shape
    sb = s // _STB
    xt = jnp.transpose(x, (0, 2, 1))
    wrt = jnp.transpose(Wr, (1, 0))
    w2t = jnp.transpose(W2, (0, 2, 1))
    w13 = jnp.concatenate([0.5 * W1, W3], axis=2)

    out = pl.pallas_call(
        _moe_body,
        grid=(b * sb,),
        in_specs=[
            pl.BlockSpec((1, d, _STB), lambda i: (i // sb, 0, i % sb)),
            pl.BlockSpec((_E, _D), lambda i: (0, 0)),
            pl.BlockSpec((_E, _D, 2 * _F), lambda i: (0, 0, 0)),
            pl.BlockSpec((_E, _D, _F), lambda i: (0, 0, 0)),
        ],
        out_specs=pl.BlockSpec((1, d, _STB), lambda i: (i // sb, 0, i % sb)),
        out_shape=jax.ShapeDtypeStruct((b, d, s), jnp.float32),
        compiler_params=pltpu.CompilerParams(
            dimension_semantics=("parallel",),
        ),
    )(xt, wrt, w13, w2t)
    return jnp.transpose(out, (0, 2, 1))
